# Initial kernel scaffold; baseline (speedup 1.0000x reference)
import jax
import jax.numpy as jnp
from jax.experimental import pallas as pl


def kernel(feats, Wgat, al, ar, Wgcn, bgcn, Wres, bres, gg, gb, ws, bs, W1, b1, g1, be1, W2, b2, edge_index):
    raise NotImplementedError("write your pallas kernel here")


# speedup vs baseline: 11.2187x; 11.2187x over previous
import functools
import math

import jax
import jax.numpy as jnp
from jax import lax
from jax.experimental import pallas as pl
from jax.experimental.pallas import tpu as pltpu
from jax.experimental.pallas import tpu_sc as plsc

N = 10000
E = 160000
D = 256
H = 2
HID = D * H
NT = 12
PH = 128
R = 3
EPS = 1e-5
ALPHA = 0.5

NC = 2
NS = 16
NW = NC * NS

EB = 128
NUM_EB = E // EB
NB_PER_TILE = (NUM_EB + NS - 1) // NS
ZR = 200
NZCH = N // ZR
NZ_PER_TILE = (NZCH + NS - 1) // NS

_MESH = dict(core_axis_name="c", subcore_axis_name="s", num_cores=NC,
             num_subcores=NS)

_i32 = jnp.int32


def _im_row(i):
    return (i, 0 * i)


def _im_fix(i):
    return (0 * i, 0 * i)


def _zero2d(ref, rows, width):
    def zloop(r, c):
        rr = ref.at[r]
        for cc in range(width // 16):
            rr[pl.ds(cc * 16, 16)] = jnp.zeros((16,), jnp.float32)
        return c
    lax.fori_loop(_i32(0), _i32(rows), zloop, _i32(0))


def _acc_zero(acc, zbuf, sid):
    for zi in range(NZ_PER_TILE):
        kch = sid + _i32(zi * NS)

        @pl.when(kch < _i32(NZCH))
        def _():
            row0 = kch * _i32(ZR)
            pltpu.sync_copy(zbuf, acc.at[pl.ds(row0, ZR), :])


def _acc_writeout(acc, out_ref, sid):
    for zi in range(NZ_PER_TILE):
        kch = sid + _i32(zi * NS)

        @pl.when(kch < _i32(NZCH))
        def _():
            row0 = kch * _i32(ZR)
            pltpu.sync_copy(acc.at[pl.ds(row0, ZR), :],
                            out_ref.at[pl.ds(row0, ZR), :])


def _sc_msg(h0, h1, h2, h3, ee4, src, dst):
    mesh = plsc.VectorSubcoreMesh(**_MESH)

    @functools.partial(
        pl.kernel, mesh=mesh,
        compiler_params=pltpu.CompilerParams(needs_layout_passes=False),
        out_type=[jax.ShapeDtypeStruct((E, 128), jnp.float32)] * 4
                 + [jax.ShapeDtypeStruct((2 * E,), jnp.float32)],
        scratch_types=[pltpu.VMEM((4 * N,), jnp.float32),
                       pltpu.VMEM((EB,), jnp.int32),
                       pltpu.VMEM((EB,), jnp.int32),
                       pltpu.VMEM((EB, 128), jnp.float32),
                       pltpu.VMEM((EB,), jnp.float32),
                       pltpu.VMEM((2 * EB,), jnp.float32),
                       pltpu.SemaphoreType.DMA])
    def k(h0_h, h1_h, h2_h, h3_h, ee_h, src_h, dst_h,
          m0, m1, m2, m3, exf,
          ee_v, src_v, dst_v, rows_v, exb, exfb, sem):
        cid = lax.axis_index("c")
        sid = lax.axis_index("s")
        h_ins = [h0_h, h1_h, h2_h, h3_h]
        m_outs = [m0, m1, m2, m3]

        pltpu.sync_copy(ee_h, ee_v)

        for job in range(5):
            owner = job % NC
            active = cid == _i32(owner)
            head = job // 2

            @pl.when(active)
            def _run():
                def bloop(j, c):
                    b = sid + j * _i32(NS)

                    @pl.when(b < _i32(NUM_EB))
                    def _():
                        base = b * _i32(EB)
                        pltpu.sync_copy(src_h.at[pl.ds(base, EB)], src_v)
                        pltpu.sync_copy(dst_h.at[pl.ds(base, EB)], dst_v)
                        if job < 4:
                            pltpu.async_copy(h_ins[job].at[src_v], rows_v,
                                             sem).wait()
                        for g in range(EB // 16):
                            s16 = src_v[pl.ds(g * 16, 16)]
                            d16 = dst_v[pl.ds(g * 16, 16)]
                            heads = (head,) if job < 4 else (0, 1)
                            for hh in heads:
                                el = plsc.load_gather(
                                    ee_v, [s16 * _i32(4) + _i32(hh)])
                                er = plsc.load_gather(
                                    ee_v, [d16 * _i32(4) + _i32(2 + hh)])
                                xx = el + er
                                ev = jnp.where(xx >= 0.0, xx, 0.2 * xx)
                                ex = jnp.exp(ev)
                                if job < 4:
                                    exb[pl.ds(g * 16, 16)] = ex
                                else:
                                    idx = ((_i32(g * 16)
                                            + lax.iota(jnp.int32, 16))
                                           * _i32(2) + _i32(hh))
                                    plsc.store_scatter(exfb, [idx], ex)
                        if job < 4:
                            def rloop(r, c2):
                                av = plsc.load_gather(
                                    exb, [jnp.full((16,), r, jnp.int32)])
                                rr = rows_v.at[r]
                                for cc in range(8):
                                    sl = pl.ds(cc * 16, 16)
                                    rr[sl] = rr[sl] * av
                                return c2
                            lax.fori_loop(_i32(0), _i32(EB), rloop, _i32(0))
                            pltpu.sync_copy(
                                rows_v, m_outs[job].at[pl.ds(base, EB), :])
                        else:
                            pltpu.sync_copy(
                                exfb, exf.at[pl.ds(base * _i32(2), 2 * EB)])

                    return c

                lax.fori_loop(_i32(0), _i32(NB_PER_TILE), bloop, _i32(0))

    return k(h0, h1, h2, h3, ee4, src, dst)


def _sc_segadd(vals, exf, dst):
    nv = len(vals)
    njobs = nv + (1 if exf is not None else 0)
    mesh = plsc.VectorSubcoreMesh(**_MESH)

    ins = list(vals) + ([exf] if exf is not None else []) + [dst]
    scratch = [pltpu.VMEM((EB,), jnp.int32),
               pltpu.VMEM((EB, 128), jnp.float32),
               pltpu.VMEM((2 * EB,), jnp.float32),
               pltpu.VMEM((ZR, 128), jnp.float32),
               pltpu.VMEM_SHARED((N, 128), jnp.float32)]

    def body(*refs):
        v_ins = refs[:nv]
        pos = nv
        exf_h = None
        if exf is not None:
            exf_h = refs[pos]
            pos += 1
        dst_h = refs[pos]
        pos += 1
        outs = refs[pos:pos + njobs]
        dst_v, rows_v, exfb, zbuf, acc = refs[pos + njobs:]
        cid = lax.axis_index("c")
        sid = lax.axis_index("s")

        _zero2d(zbuf, ZR, 128)

        for job in range(njobs):
            active = cid == _i32(job % NC)
            is_den = job == nv

            @pl.when(active)
            def _zero():
                _acc_zero(acc, zbuf, sid)
                if is_den:
                    _zero2d(rows_v, EB, 128)

            plsc.subcore_barrier()

            @pl.when(active)
            def _accumulate():
                def bloop(j, c):
                    b = sid + j * _i32(NS)

                    @pl.when(b < _i32(NUM_EB))
                    def _():
                        base = b * _i32(EB)
                        pltpu.sync_copy(dst_h.at[pl.ds(base, EB)], dst_v)
                        if is_den:
                            pltpu.sync_copy(
                                exf_h.at[pl.ds(base * _i32(2), 2 * EB)],
                                exfb)
                            for g in range(EB // 16):
                                ridx = (_i32(g * 16)
                                        + lax.iota(jnp.int32, 16))
                                for hh in range(2):
                                    ex = plsc.load_gather(
                                        exfb,
                                        [ridx * _i32(2) + _i32(hh)])
                                    cidx = jnp.full((16,), hh, jnp.int32)
                                    plsc.store_scatter(rows_v, [ridx, cidx],
                                                       ex)
                        else:
                            pltpu.sync_copy(
                                v_ins[job].at[pl.ds(base, EB), :], rows_v)
                        pltpu.sync_copy(rows_v, acc.at[dst_v], add=True)

                    return c

                lax.fori_loop(_i32(0), _i32(NB_PER_TILE), bloop, _i32(0))

            plsc.subcore_barrier()

            @pl.when(active)
            def _writeout():
                _acc_writeout(acc, outs[job], sid)

            plsc.subcore_barrier()

    k = functools.partial(
        pl.kernel, mesh=mesh,
        compiler_params=pltpu.CompilerParams(needs_layout_passes=False),
        out_type=[jax.ShapeDtypeStruct((N, 128), jnp.float32)] * njobs,
        scratch_types=scratch)(body)
    return k(*ins)


def _sc_zgather(z0, z1, src):
    mesh = plsc.VectorSubcoreMesh(**_MESH)

    @functools.partial(
        pl.kernel, mesh=mesh,
        compiler_params=pltpu.CompilerParams(needs_layout_passes=False),
        out_type=[jax.ShapeDtypeStruct((E, 128), jnp.float32)] * 2,
        scratch_types=[pltpu.VMEM((EB,), jnp.int32),
                       pltpu.VMEM((EB, 128), jnp.float32),
                       pltpu.SemaphoreType.DMA])
    def k(z0_h, z1_h, src_h, o0, o1, src_v, rows_v, sem):
        cid = lax.axis_index("c")
        sid = lax.axis_index("s")
        z_ins = [z0_h, z1_h]
        outs = [o0, o1]

        for job in range(2):
            active = cid == _i32(job % NC)

            @pl.when(active)
            def _run():
                def bloop(j, c):
                    b = sid + j * _i32(NS)

                    @pl.when(b < _i32(NUM_EB))
                    def _():
                        base = b * _i32(EB)
                        pltpu.sync_copy(src_h.at[pl.ds(base, EB)], src_v)
                        pltpu.async_copy(z_ins[job].at[src_v], rows_v,
                                         sem).wait()
                        pltpu.sync_copy(rows_v,
                                        outs[job].at[pl.ds(base, EB), :])

                    return c

                lax.fori_loop(_i32(0), _i32(NB_PER_TILE), bloop, _i32(0))

    return k(z0, z1, src)


def _sc_deg(idx0, idx1):
    mesh = plsc.VectorSubcoreMesh(**_MESH)

    @functools.partial(
        pl.kernel, mesh=mesh,
        out_type=[jax.ShapeDtypeStruct((N, 128), jnp.float32)] * 2,
        scratch_types=[pltpu.VMEM((EB,), jnp.int32),
                       pltpu.VMEM((EB, 128), jnp.float32),
                       pltpu.VMEM((ZR, 128), jnp.float32),
                       pltpu.VMEM_SHARED((N, 128), jnp.float32)])
    def k(i0_h, i1_h, d0, d1, idx_v, rows_v, zbuf, acc):
        cid = lax.axis_index("c")
        sid = lax.axis_index("s")
        idx_ins = [i0_h, i1_h]
        outs = [d0, d1]

        _zero2d(zbuf, ZR, 128)

        def oloop(r, c):
            rr = rows_v.at[r]
            for cc in range(8):
                rr[pl.ds(cc * 16, 16)] = jnp.ones((16,), jnp.float32)
            return c
        lax.fori_loop(_i32(0), _i32(EB), oloop, _i32(0))

        for job in range(2):
            active = cid == _i32(job % NC)

            @pl.when(active)
            def _zero():
                _acc_zero(acc, zbuf, sid)

            plsc.subcore_barrier()

            @pl.when(active)
            def _accumulate():
                def bloop(j, c):
                    b = sid + j * _i32(NS)

                    @pl.when(b < _i32(NUM_EB))
                    def _():
                        base = b * _i32(EB)
                        pltpu.sync_copy(idx_ins[job].at[pl.ds(base, EB)],
                                        idx_v)
                        pltpu.sync_copy(rows_v, acc.at[idx_v], add=True)

                    return c

                lax.fori_loop(_i32(0), _i32(NB_PER_TILE), bloop, _i32(0))

            plsc.subcore_barrier()

            @pl.when(active)
            def _writeout():
                _acc_writeout(acc, outs[job], sid)

            plsc.subcore_barrier()

    return k(idx0, idx1)


def _mm_gat(x, Wg, am, bn=1000):
    def kern(x_ref, w_ref, a_ref, o0, o1, o2, o3, oe):
        h = jnp.dot(x_ref[...], w_ref[...],
                    preferred_element_type=jnp.float32)
        for j, o in enumerate((o0, o1, o2, o3)):
            o[...] = h[:, j * 128:(j + 1) * 128]
        oe[...] = jnp.dot(h, a_ref[...], preferred_element_type=jnp.float32)

    return pl.pallas_call(
        kern,
        grid=(N // bn,),
        in_specs=[pl.BlockSpec((bn, D), _im_row),
                  pl.BlockSpec((D, HID), _im_fix),
                  pl.BlockSpec((HID, 128), _im_fix)],
        out_specs=[pl.BlockSpec((bn, 128), _im_row)] * 5,
        out_shape=[jax.ShapeDtypeStruct((N, 128), jnp.float32)] * 5,
    )(x, Wg, am)


def _mm_post(us, den, w, bias=None, scale=None, out_relu=False, split_out=1,
             bn=1000):
    m = w.shape[1]
    ins = list(us) + [den, w]
    specs = [pl.BlockSpec((bn, 128), _im_row)] * 4 + \
            [pl.BlockSpec((bn, 128), _im_row),
             pl.BlockSpec((HID, m), _im_fix)]
    if bias is not None:
        ins.append(bias.reshape(1, m))
        specs.append(pl.BlockSpec((1, m), _im_fix))
    if scale is not None:
        ins.append(scale)
        specs.append(pl.BlockSpec((bn, 128), _im_row))
    has_b, has_s = bias is not None, scale is not None

    def kern(*refs):
        u_refs = refs[:4]
        dn_ref = refs[4]
        w_ref = refs[5]
        pos = 6
        b_ref = s_ref = None
        if has_b:
            b_ref = refs[pos]
            pos += 1
        if has_s:
            s_ref = refs[pos]
            pos += 1
        o_refs = refs[pos:]

        d0 = 1.0 / jnp.maximum(dn_ref[:, 0:1], 1e-9)
        d1 = 1.0 / jnp.maximum(dn_ref[:, 1:2], 1e-9)
        xv = jnp.concatenate(
            [jnp.maximum(u_refs[j][...], 0.0) * (d0 if j < 2 else d1)
             for j in range(4)], axis=1)
        if s_ref is not None:
            xv = xv * lax.rsqrt(jnp.maximum(s_ref[:, 0:1], 1.0))
        acc = jnp.dot(xv, w_ref[...], preferred_element_type=jnp.float32)
        if b_ref is not None:
            acc = acc + b_ref[...]
        if out_relu:
            acc = jnp.maximum(acc, 0.0)
        if split_out == 1:
            o_refs[0][...] = acc
        else:
            mc = m // split_out
            for j in range(split_out):
                o_refs[j][...] = acc[:, j * mc:(j + 1) * mc]

    mc = m // split_out
    return pl.pallas_call(
        kern,
        grid=(N // bn,),
        in_specs=specs,
        out_specs=[pl.BlockSpec((bn, mc), _im_row)] * split_out,
        out_shape=[jax.ShapeDtypeStruct((N, mc), jnp.float32)] * split_out,
    )(*ins)


def _round_finish(aggs, res, feats, deg_in, bgcn_i, gg_i, gb_i, bn=1000):
    inv = 1.0 / math.sqrt(1.0 + EPS)

    def kern(a0_ref, a1_ref, r_ref, f_ref, d_ref, bg_ref,
             gg_ref, gb_ref, o_ref):
        nin = lax.rsqrt(jnp.maximum(d_ref[:, 0:1], 1.0))
        agg = jnp.concatenate([a0_ref[...], a1_ref[...]], axis=1)
        agg = jnp.maximum(agg * nin + bg_ref[...], 0.0)
        out = (agg + r_ref[...]) * inv * gg_ref[...] + gb_ref[...]
        o_ref[...] = ALPHA * out + (1.0 - ALPHA) * f_ref[...]

    return pl.pallas_call(
        kern,
        grid=(N // bn,),
        in_specs=[pl.BlockSpec((bn, 128), _im_row)] * 2 +
                 [pl.BlockSpec((bn, D), _im_row),
                  pl.BlockSpec((bn, D), _im_row),
                  pl.BlockSpec((bn, 128), _im_row),
                  pl.BlockSpec((1, D), _im_fix),
                  pl.BlockSpec((1, D), _im_fix),
                  pl.BlockSpec((1, D), _im_fix)],
        out_specs=pl.BlockSpec((bn, D), _im_row),
        out_shape=jax.ShapeDtypeStruct((N, D), jnp.float32),
    )(*aggs, res, feats, deg_in, bgcn_i.reshape(1, D),
      gg_i.reshape(1, D), gb_i.reshape(1, D))


def _readout(feats, wsr, bs, bn=1000):
    def kern(f_ref, w_ref, b_ref, sum_ref, max_ref):
        i = pl.program_id(0)
        f = f_ref[...]
        s = jnp.sum(f * w_ref[...], axis=1, keepdims=True) + b_ref[0, 0]
        wgt = jax.nn.sigmoid(s)
        psum = jnp.sum(wgt * f, axis=0, keepdims=True)
        pmax = jnp.max(f, axis=0, keepdims=True)

        @pl.when(i == 0)
        def _():
            sum_ref[...] = psum
            max_ref[...] = pmax

        @pl.when(i != 0)
        def _():
            sum_ref[...] += psum
            max_ref[...] = jnp.maximum(max_ref[...], pmax)

    return pl.pallas_call(
        kern,
        grid=(N // bn,),
        in_specs=[pl.BlockSpec((bn, D), _im_row),
                  pl.BlockSpec((1, D), _im_fix),
                  pl.BlockSpec((1, 1), _im_fix)],
        out_specs=[pl.BlockSpec((1, D), _im_fix),
                   pl.BlockSpec((1, D), _im_fix)],
        out_shape=[jax.ShapeDtypeStruct((1, D), jnp.float32),
                   jax.ShapeDtypeStruct((1, D), jnp.float32)],
    )(feats, wsr, bs.reshape(1, 1))


def _mlp(hg, W1, b1, g1, be1, W2p, b2p):
    inv = 1.0 / math.sqrt(1.0 + EPS)

    def kern(h_ref, w1_ref, b1_ref, g1_ref, be1_ref, w2_ref, b2_ref, o_ref):
        p = jnp.dot(h_ref[...], w1_ref[...],
                    preferred_element_type=jnp.float32)
        p = jnp.maximum(p + b1_ref[...], 0.0)
        p = p * inv * g1_ref[...] + be1_ref[...]
        o_ref[...] = jnp.dot(p, w2_ref[...],
                             preferred_element_type=jnp.float32) + b2_ref[...]

    return pl.pallas_call(
        kern,
        out_shape=jax.ShapeDtypeStruct((1, 128), jnp.float32),
    )(hg, W1, b1.reshape(1, PH), g1.reshape(1, PH), be1.reshape(1, PH),
      W2p, b2p)


def kernel(feats, Wgat, al, ar, Wgcn, bgcn, Wres, bres, gg, gb, ws, bs,
           W1, b1, g1, be1, W2, b2, edge_index):
    feats = feats.astype(jnp.float32)
    src = edge_index[0].astype(jnp.int32)
    dst = edge_index[1].astype(jnp.int32)

    deg_in, deg_out = _sc_deg(dst, src)

    x = feats
    for i in range(R):
        am = jnp.zeros((HID, 128), jnp.float32)
        am = am.at[0:D, 0].set(al[i, 0]).at[D:HID, 1].set(al[i, 1])
        am = am.at[0:D, 2].set(ar[i, 0]).at[D:HID, 3].set(ar[i, 1])

        h0, h1, h2, h3, ee128 = _mm_gat(x, Wgat[i], am)
        ee4 = ee128[:, 0:4].reshape(-1)

        m0, m1, m2, m3, exf = _sc_msg(h0, h1, h2, h3, ee4, src, dst)
        u0, u1, u2, u3, den = _sc_segadd([m0, m1, m2, m3], exf, dst)
        us = [u0, u1, u2, u3]

        z0, z1 = _mm_post(us, den, Wgcn[i], scale=deg_out, split_out=2)
        zs0, zs1 = _sc_zgather(z0, z1, src)
        aggs = _sc_segadd([zs0, zs1], None, dst)
        res, = _mm_post(us, den, Wres[i], bias=bres[i], out_relu=True)
        x = _round_finish(aggs, res, x, deg_in, bgcn[i], gg[i], gb[i])

    h_sum, h_max = _readout(x, ws.reshape(1, D).astype(jnp.float32),
                            bs.astype(jnp.float32))
    hg = jnp.concatenate([h_sum, h_max], axis=1)
    W2p = jnp.zeros((PH, 128), jnp.float32).at[:, 0:NT].set(W2)
    b2p = jnp.zeros((1, 128), jnp.float32).at[0, 0:NT].set(b2)
    out128 = _mlp(hg, W1, b1, g1, be1, W2p, b2p)
    return out128[:, 0:NT]
